# Initial kernel scaffold; baseline (speedup 1.0000x reference)
import functools

import jax
import jax.numpy as jnp
from jax import lax
from jax.experimental import pallas as pl
from jax.experimental.pallas import tpu as pltpu
from jax.experimental.pallas import tpu_sc as plsc

NC = 2
NS = 16
LANES = 16
CHUNK = 128


def _make(B, H, D, V):
    NW = NC * NS
    G = 16
    assert B % (NW * G) == 0
    NG = B // (NW * G)
    assert (G * H) % CHUNK == 0
    NCH = (G * H) // CHUNK
    assert D == 2 * LANES

    mesh = plsc.VectorSubcoreMesh(core_axis_name="c", subcore_axis_name="s")

    @functools.partial(
        pl.kernel,
        out_type=jax.ShapeDtypeStruct((B, D), jnp.float32),
        mesh=mesh,
        scratch_types=[
            pltpu.VMEM((NCH, CHUNK), jnp.int32),
            pltpu.VMEM((G * H, D), jnp.float32),
            pltpu.VMEM((G, D), jnp.float32),
            pltpu.VMEM((D,), jnp.float32),
            pltpu.SemaphoreType.DMA,
        ],
    )
    def run(idx_hbm, table_hbm, bias_hbm, out_hbm, idx_v, rows_v, out_v,
            bias_v, sem):
        wid = lax.axis_index("s") * NC + lax.axis_index("c")

        pltpu.sync_copy(bias_hbm, bias_v)
        bias_lo = bias_v[pl.ds(0, LANES)]
        bias_hi = bias_v[pl.ds(LANES, LANES)]

        def group(g, carry):
            gg = wid * NG + g
            pltpu.sync_copy(idx_hbm.at[pl.ds(gg * NCH, NCH)], idx_v)
            descs = [
                pltpu.async_copy(
                    table_hbm.at[idx_v.at[j]],
                    rows_v.at[pl.ds(j * CHUNK, CHUNK)],
                    sem,
                )
                for j in range(NCH)
            ]
            for d in descs:
                d.wait()

            for b in range(G):
                base = b * H

                def rstep(i, accs):
                    res = []
                    for u in range(8):
                        r = base + i * 8 + u
                        res.append(accs[2 * u] + rows_v[r, pl.ds(0, LANES)])
                        res.append(
                            accs[2 * u + 1] + rows_v[r, pl.ds(LANES, LANES)])
                    return tuple(res)

                zero = jnp.zeros((LANES,), jnp.float32)
                accs = lax.fori_loop(0, H // 8, rstep, (zero,) * 16)
                lo = ((accs[0] + accs[2]) + (accs[4] + accs[6])) + \
                     ((accs[8] + accs[10]) + (accs[12] + accs[14]))
                hi = ((accs[1] + accs[3]) + (accs[5] + accs[7])) + \
                     ((accs[9] + accs[11]) + (accs[13] + accs[15]))
                out_v[b, pl.ds(0, LANES)] = lo + bias_lo
                out_v[b, pl.ds(LANES, LANES)] = hi + bias_hi

            pltpu.sync_copy(out_v, out_hbm.at[pl.ds(gg * G, G)])
            return carry

        lax.fori_loop(0, NG, group, 0)

    return run


def kernel(inputs, table, bias):
    B, H = inputs.shape
    V, D = table.shape
    idx = inputs.reshape(-1, CHUNK).astype(jnp.int32)
    return _make(B, H, D, V)(idx, table, bias)


# speedup vs baseline: 14.1105x; 14.1105x over previous
import functools

import jax
import jax.numpy as jnp
from jax import lax
from jax.experimental import pallas as pl
from jax.experimental.pallas import tpu as pltpu
from jax.experimental.pallas import tpu_sc as plsc

NC = 2
NS = 16
LANES = 16
CHUNK = 128


def _make(B, H, D, V):
    NW = NC * NS
    G = 16
    assert B % (NW * G) == 0
    NG = B // (NW * G)
    assert (G * H) % CHUNK == 0
    NCH = (G * H) // CHUNK
    assert D == 2 * LANES

    mesh = plsc.VectorSubcoreMesh(core_axis_name="c", subcore_axis_name="s")

    @functools.partial(
        pl.kernel,
        out_type=jax.ShapeDtypeStruct((B, D), jnp.float32),
        mesh=mesh,
        scratch_types=[
            pltpu.VMEM((G * H,), jnp.int32),
            pltpu.VMEM((G * H, D), jnp.float32),
            pltpu.VMEM((G, D), jnp.float32),
            pltpu.VMEM((D,), jnp.float32),
            pltpu.SemaphoreType.DMA,
        ],
        compiler_params=pltpu.CompilerParams(use_tc_tiling_on_sc=False),
    )
    def run(idx_hbm, table_hbm, bias_hbm, out_hbm, idx_v, rows_v, out_v,
            bias_v, sem):
        wid = lax.axis_index("s") * NC + lax.axis_index("c")

        pltpu.sync_copy(bias_hbm, bias_v)
        bias_lo = bias_v[pl.ds(0, LANES)]
        bias_hi = bias_v[pl.ds(LANES, LANES)]

        def group(g, carry):
            gg = wid * NG + g
            pltpu.sync_copy(idx_hbm.at[pl.ds(gg * G * H, G * H)], idx_v)
            descs = [
                pltpu.async_copy(
                    table_hbm.at[idx_v.at[pl.ds(j * CHUNK, CHUNK)]],
                    rows_v.at[pl.ds(j * CHUNK, CHUNK)],
                    sem,
                )
                for j in range(NCH)
            ]
            for d in descs:
                d.wait()

            for b in range(G):
                base = b * H

                def rstep(i, accs):
                    res = []
                    for u in range(8):
                        r = base + i * 8 + u
                        res.append(accs[2 * u] + rows_v[r, pl.ds(0, LANES)])
                        res.append(
                            accs[2 * u + 1] + rows_v[r, pl.ds(LANES, LANES)])
                    return tuple(res)

                zero = jnp.zeros((LANES,), jnp.float32)
                accs = lax.fori_loop(0, H // 8, rstep, (zero,) * 16)
                lo = ((accs[0] + accs[2]) + (accs[4] + accs[6])) + \
                     ((accs[8] + accs[10]) + (accs[12] + accs[14]))
                hi = ((accs[1] + accs[3]) + (accs[5] + accs[7])) + \
                     ((accs[9] + accs[11]) + (accs[13] + accs[15]))
                out_v[b, pl.ds(0, LANES)] = lo + bias_lo
                out_v[b, pl.ds(LANES, LANES)] = hi + bias_hi

            pltpu.sync_copy(out_v, out_hbm.at[pl.ds(gg * G, G)])
            return carry

        lax.fori_loop(0, NG, group, 0)

    return run


def kernel(inputs, table, bias):
    B, H = inputs.shape
    V, D = table.shape
    idx = inputs.reshape(-1).astype(jnp.int32)
    return _make(B, H, D, V)(idx, table, bias)
